# baseline (speedup 1.0000x reference)
import functools
import jax
import jax.numpy as jnp
from jax import lax
from jax.experimental import pallas as pl
from jax.experimental.pallas import tpu as pltpu
from jax.experimental.pallas import tpu_sc as plsc

_BN = 80
_R = _BN * 49
_BNU = 1000
_BE = 4000


def _mish(v):
    return v * jnp.tanh(jax.nn.softplus(v))


def _ln(v, g, b):
    m = v.mean(-1, keepdims=True)
    s = ((v - m) ** 2).mean(-1, keepdims=True)
    return (v - m) / jnp.sqrt(s + 1e-5) * g + b


def _dot(a, b):
    return jnp.dot(a, b, preferred_element_type=jnp.float32)


def _bspec(a):
    nd = a.ndim
    return pl.BlockSpec(a.shape, lambda i, _nd=nd: (0,) * _nd)


def _node_body(x_ref, gp_ref, maw1, mab1, maw2, mab2, mag, mabe,
               new1, neb1, neg1, nebe1, new2, neb2, neg2, nebe2,
               w1, b1, w2, b2, w3, b3, p0w,
               nf_ref, p0_ref):
    x = x_ref[...]
    h = jnp.maximum(_dot(x, maw1[...]) + mab1[...], 0.0)
    h = _dot(h, maw2[...]) + mab2[...]
    ma = mag[...] * h + mabe[...]
    nf = jnp.maximum(_ln(_dot(ma, new1[...]) + neb1[...], neg1[...], nebe1[...]), 0.0)
    nf = _mish(_ln(_dot(nf, new2[...]) + neb2[...], neg2[...], nebe2[...]))

    X = gp_ref[...]
    pos = lax.broadcasted_iota(jnp.int32, (_R, 1), 0) % 49
    hh, ww = pos // 7, pos % 7

    def conv(Xi, W, b):
        cols = []
        for dh in (-1, 0, 1):
            for dw in (-1, 0, 1):
                off = dh * 7 + dw
                if off == 0:
                    cols.append(Xi)
                    continue
                m = (hh + dh >= 0) & (hh + dh < 7) & (ww + dw >= 0) & (ww + dw < 7)
                cols.append(jnp.where(m, jnp.roll(Xi, -off, axis=0), 0.0))
        return _mish(_dot(jnp.concatenate(cols, axis=1), W[...]) + b[...])

    g1 = conv(X, w1, b1)
    g2 = conv(g1, w2, b2)
    g3 = conv(g2, w3, b3)
    gf = g3.reshape(_BN, 49, 64).mean(axis=1)

    nfeat = jnp.concatenate([nf, gf], axis=1)
    nf_ref[...] = nfeat
    p0_ref[...] = _dot(nfeat, p0w[...])


def _edge_body(ea_ref, w1, b1, g1, be1, w2, b2, g2, be2,
               q0w, q0b, q1w, q1b, q0_ref, q1_ref):
    ea = ea_ref[...]
    ef = jnp.maximum(_ln(_dot(ea, w1[...]) + b1[...], g1[...], be1[...]), 0.0)
    ef = _mish(_ln(_dot(ef, w2[...]) + b2[...], g2[...], be2[...]))
    q0_ref[...] = _dot(ef, q0w[...]) + q0b[...]
    q1_ref[...] = _dot(ef, q1w[...]) + q1b[...]


def _upd_body(has_p, hc_ref, ag_ref, uwn, uwa, ub, g, be, pw, hc_out, *p_out):
    hc = hc_ref[...]
    ag = ag_ref[...]
    agg = ag[0] + ag[1]
    u = _mish(_dot(hc, uwn[...]) + _dot(agg, uwa[...]) + ub[...])
    hn = _ln(hc + u, g[...], be[...])
    hc_out[...] = hn
    if has_p:
        p_out[0][...] = _dot(hn, pw[...])


def _segsum_body(ids_ref, hc_ref, sum_ref, cnt_ref):
    i = pl.program_id(0)
    oh = (ids_ref[...] == lax.broadcasted_iota(jnp.int32, (1, 16), 1)).astype(jnp.float32)
    ps = lax.dot_general(oh, hc_ref[...], (((0,), (0,)), ((), ())),
                         preferred_element_type=jnp.float32)
    pc = oh.sum(axis=0, keepdims=True)

    @pl.when(i == 0)
    def _():
        sum_ref[...] = jnp.zeros_like(sum_ref)
        cnt_ref[...] = jnp.zeros_like(cnt_ref)

    sum_ref[...] += ps
    cnt_ref[...] += pc


def _heads_body(hc_ref, ids_ref, sum_ref, cnt_ref,
                sgw1, sgb1, sgg1, sgbe1, sgw2, sgb2,
                qw1, qb1, qg1, qbe1, qw2, qb2, qg2, qbe2,
                kw1, kb1, kg1, kbe1, kw2, kb2, kg2, kbe2,
                bw1, bb1, bg1, bbe1, bw2, bb2, bg2, bbe2, bw3, bb3,
                seg_ref, q_ref, k_ref, bt_ref, lg_ref):
    cnt = jnp.maximum(cnt_ref[...], 1.0)
    oh = (ids_ref[...] == lax.broadcasted_iota(jnp.int32, (1, 16), 1)).astype(jnp.float32)
    gmn = _dot(oh / cnt, sum_ref[...])
    lg = jnp.concatenate([hc_ref[...], gmn], axis=1)
    lg_ref[...] = lg

    seg = _mish(_ln(_dot(lg, sgw1[...]) + sgb1[...], sgg1[...], sgbe1[...]))
    seg_ref[...] = _dot(seg, sgw2[...]) + sgb2[...]

    q = _mish(_ln(_dot(lg, qw1[...]) + qb1[...], qg1[...], qbe1[...]))
    q_ref[...] = _ln(_dot(q, qw2[...]) + qb2[...], qg2[...], qbe2[...])

    k = _mish(_ln(_dot(lg, kw1[...]) + kb1[...], kg1[...], kbe1[...]))
    k_ref[...] = _ln(_dot(k, kw2[...]) + kb2[...], kg2[...], kbe2[...])

    bt = jax.nn.gelu(_ln(_dot(lg, bw1[...]) + bb1[...], bg1[...], bbe1[...]))
    bt = jax.nn.gelu(_ln(_dot(bt, bw2[...]) + bb2[...], bg2[...], bbe2[...]))
    bt_ref[...] = _dot(bt, bw3[...]) + bb3[...]


_NC, _NS = 2, 16
_NW = _NC * _NS
_CE = 80
_ZR = 128


def _sc_mish(v):
    w = 1.0 + jnp.exp(jnp.minimum(v, 30.0))
    w2 = w * w
    return v * (w2 - 1.0) / (w2 + 1.0)


def _sc_msg_body(npad, epw, nchunks,
                 P_hbm, Q_hbm, src_hbm, dst_hbm, out_hbm,
                 isrc, idst, rows, qrow, mbuf, zbuf, agg_sh, sem):
    cid = lax.axis_index("c")
    sid = lax.axis_index("s")
    wid = sid * _NC + cid
    rpt = npad // _NS

    def zb(i, _):
        for j in range(8):
            zbuf[i, pl.ds(j * 16, 16)] = jnp.zeros((16,), jnp.float32)
        return _
    lax.fori_loop(0, _ZR, zb, None)
    for c in range(rpt // _ZR):
        pltpu.sync_copy(zbuf, agg_sh.at[pl.ds(sid * rpt + c * _ZR, _ZR)])
    plsc.subcore_barrier()

    def chunk(kk, _):
        eb = wid * epw + kk * _CE
        pltpu.sync_copy(src_hbm.at[pl.ds(eb, _CE)], isrc)
        pltpu.async_copy(P_hbm.at[isrc], rows, sem).wait()
        pltpu.sync_copy(Q_hbm.at[pl.ds(eb, _CE)], qrow)

        def ebody(i, _):
            for j in range(8):
                v = rows[i, pl.ds(j * 16, 16)] + qrow[i, pl.ds(j * 16, 16)]
                mbuf[i, pl.ds(j * 16, 16)] = _sc_mish(v)
            return _
        lax.fori_loop(0, _CE, ebody, None)

        pltpu.sync_copy(dst_hbm.at[pl.ds(eb, _CE)], idst)
        pltpu.sync_copy(mbuf, agg_sh.at[idst], add=True)
        return _
    lax.fori_loop(0, nchunks, chunk, None)

    plsc.subcore_barrier()
    for c in range(rpt // _ZR):
        r0 = sid * rpt + c * _ZR
        pltpu.sync_copy(agg_sh.at[pl.ds(r0, _ZR)], zbuf)
        pltpu.sync_copy(zbuf, out_hbm.at[pl.ds(cid * npad + r0, _ZR)])


def _msg_pass(P, Q, srci, dsti, n):
    E = srci.shape[0]
    epw = E // _NW
    nchunks = epw // _CE
    npad = ((n + 2047) // 2048) * 2048
    mesh = plsc.VectorSubcoreMesh(core_axis_name="c", subcore_axis_name="s")
    out = pl.kernel(
        functools.partial(_sc_msg_body, npad, epw, nchunks),
        out_type=jax.ShapeDtypeStruct((2 * npad, 128), jnp.float32),
        mesh=mesh,
        scratch_types=[
            pltpu.VMEM((_CE,), jnp.int32),
            pltpu.VMEM((_CE,), jnp.int32),
            pltpu.VMEM((_CE, 128), jnp.float32),
            pltpu.VMEM((_CE, 128), jnp.float32),
            pltpu.VMEM((_CE, 128), jnp.float32),
            pltpu.VMEM((_ZR, 128), jnp.float32),
            pltpu.VMEM_SHARED((npad, 128), jnp.float32),
            pltpu.SemaphoreType.DMA,
        ],
    )(P, Q, srci, dsti)
    return out.reshape(2, npad, 128)


def _sc_inst_body(epw, nchunks,
                  q_hbm, k_hbm, src_hbm, dst_hbm, out_hbm,
                  isrc, idst, qrows, krows, obuf, sem):
    cid = lax.axis_index("c")
    sid = lax.axis_index("s")
    wid = sid * _NC + cid

    def chunk(kk, _):
        eb = wid * epw + kk * _CE
        pltpu.sync_copy(src_hbm.at[pl.ds(eb, _CE)], isrc)
        pltpu.sync_copy(dst_hbm.at[pl.ds(eb, _CE)], idst)
        pltpu.async_copy(q_hbm.at[isrc], qrows, sem).wait()
        pltpu.async_copy(k_hbm.at[idst], krows, sem).wait()

        def ebody(i, _):
            acc = jnp.zeros((16,), jnp.float32)
            for j in range(16):
                acc = acc + qrows[i, pl.ds(j * 16, 16)] * krows[i, pl.ds(j * 16, 16)]
            obuf[i] = acc
            return _
        lax.fori_loop(0, _CE, ebody, None)

        pltpu.sync_copy(obuf, out_hbm.at[pl.ds(eb, _CE)])
        return _
    lax.fori_loop(0, nchunks, chunk, None)


def _inst_fin_body(pp_ref, out_ref):
    out_ref[...] = jax.nn.sigmoid(pp_ref[...].sum(axis=-1, keepdims=True))


def _inst_head(q, k, srci, dsti):
    E = srci.shape[0]
    epw = E // _NW
    nchunks = epw // _CE
    mesh = plsc.VectorSubcoreMesh(core_axis_name="c", subcore_axis_name="s")
    pp = pl.kernel(
        functools.partial(_sc_inst_body, epw, nchunks),
        out_type=jax.ShapeDtypeStruct((E, 16), jnp.float32),
        mesh=mesh,
        scratch_types=[
            pltpu.VMEM((_CE,), jnp.int32),
            pltpu.VMEM((_CE,), jnp.int32),
            pltpu.VMEM((_CE, 256), jnp.float32),
            pltpu.VMEM((_CE, 256), jnp.float32),
            pltpu.VMEM((_CE, 16), jnp.float32),
            pltpu.SemaphoreType.DMA,
        ],
    )(q, k, srci, dsti)
    inst = pl.pallas_call(
        _inst_fin_body,
        grid=(E // _BE,),
        in_specs=[pl.BlockSpec((_BE, 16), lambda i: (i, 0))],
        out_specs=pl.BlockSpec((_BE, 1), lambda i: (i, 0)),
        out_shape=jax.ShapeDtypeStruct((E, 1), jnp.float32),
    )(pp)
    return inst.reshape(E)


def kernel(x, grid, edge_attr, edge_index, node_graph_ids, params):
    p = params
    N = x.shape[0]
    E = edge_attr.shape[0]
    src, dst = edge_index[0], edge_index[1]

    r2 = lambda a: a.reshape(1, -1)

    gp = grid.transpose(0, 2, 3, 1).reshape(N * 49, grid.shape[1])

    def cw(w, g):
        return (w * g[:, None, None, None]).transpose(2, 3, 1, 0).reshape(-1, w.shape[0])

    def cb(b, g, bb):
        return (b * g + bb).reshape(1, -1)

    consts_a = [
        p['ma_w1'], r2(p['ma_b1']), p['ma_w2'], r2(p['ma_b2']), r2(p['ma_g']), r2(p['ma_be']),
        p['ne_w1'], r2(p['ne_b1']), r2(p['ne_g1']), r2(p['ne_be1']),
        p['ne_w2'], r2(p['ne_b2']), r2(p['ne_g2']), r2(p['ne_be2']),
        cw(p['c1_w'], p['bn1_g']), cb(p['c1_b'], p['bn1_g'], p['bn1_b']),
        cw(p['c2_w'], p['bn2_g']), cb(p['c2_b'], p['bn2_g'], p['bn2_b']),
        cw(p['c3_w'], p['bn3_g']), cb(p['c3_b'], p['bn3_g'], p['bn3_b']),
        p['g0_mw'][:128],
    ]
    nsteps = N // _BN
    nfeat, P0 = pl.pallas_call(
        _node_body,
        grid=(nsteps,),
        in_specs=[pl.BlockSpec((_BN, x.shape[1]), lambda i: (i, 0)),
                  pl.BlockSpec((_R, gp.shape[1]), lambda i: (i, 0))]
                 + [_bspec(a) for a in consts_a],
        out_specs=[pl.BlockSpec((_BN, 128), lambda i: (i, 0)),
                   pl.BlockSpec((_BN, 128), lambda i: (i, 0))],
        out_shape=[jax.ShapeDtypeStruct((N, 128), jnp.float32),
                   jax.ShapeDtypeStruct((N, 128), jnp.float32)],
    )(x, gp, *consts_a)

    consts_b = [
        p['ee_w1'], r2(p['ee_b1']), r2(p['ee_g1']), r2(p['ee_be1']),
        p['ee_w2'], r2(p['ee_b2']), r2(p['ee_g2']), r2(p['ee_be2']),
        p['g0_mw'][128:], r2(p['g0_mb']), p['g1_mw'][128:], r2(p['g1_mb']),
    ]
    Q0, Q1 = pl.pallas_call(
        _edge_body,
        grid=(E // _BE,),
        in_specs=[pl.BlockSpec((_BE, edge_attr.shape[1]), lambda i: (i, 0))]
                 + [_bspec(a) for a in consts_b],
        out_specs=[pl.BlockSpec((_BE, 128), lambda i: (i, 0)),
                   pl.BlockSpec((_BE, 128), lambda i: (i, 0))],
        out_shape=[jax.ShapeDtypeStruct((E, 128), jnp.float32),
                   jax.ShapeDtypeStruct((E, 128), jnp.float32)],
    )(edge_attr, *consts_b)

    def update(hc, aggpair, i, pw, has_p):
        consts = [p['g%d_uw' % i][:128], p['g%d_uw' % i][128:], r2(p['g%d_ub' % i]),
                  r2(p['g%d_g' % i]), r2(p['g%d_be' % i]), pw]
        outs = [jax.ShapeDtypeStruct((N, 128), jnp.float32)]
        ospec = [pl.BlockSpec((_BNU, 128), lambda i: (i, 0))]
        if has_p:
            outs = outs * 2
            ospec = ospec * 2
        return pl.pallas_call(
            functools.partial(_upd_body, has_p),
            grid=(N // _BNU,),
            in_specs=[pl.BlockSpec((_BNU, 128), lambda i: (i, 0)),
                      pl.BlockSpec((2, _BNU, 128), lambda i: (0, i, 0))]
                     + [_bspec(a) for a in consts],
            out_specs=ospec,
            out_shape=outs,
        )(hc, aggpair, *consts)

    agg0 = _msg_pass(P0, Q0, src, dst, N)
    hc1, P1 = update(nfeat, agg0, 0, p['g1_mw'][:128], True)
    agg1 = _msg_pass(P1, Q1, src, dst, N)
    (hc2,) = update(hc1, agg1, 1, p['g1_mw'][:128], False)

    ids2 = node_graph_ids.reshape(N, 1)
    sums, cnts = pl.pallas_call(
        _segsum_body,
        grid=(N // _BNU,),
        in_specs=[pl.BlockSpec((_BNU, 1), lambda i: (i, 0)),
                  pl.BlockSpec((_BNU, 128), lambda i: (i, 0))],
        out_specs=[pl.BlockSpec((16, 128), lambda i: (0, 0)),
                   pl.BlockSpec((1, 16), lambda i: (0, 0))],
        out_shape=[jax.ShapeDtypeStruct((16, 128), jnp.float32),
                   jax.ShapeDtypeStruct((1, 16), jnp.float32)],
    )(ids2, hc2)

    consts_h = [
        p['sg_w1'], r2(p['sg_b1']), r2(p['sg_g1']), r2(p['sg_be1']), p['sg_w2'], r2(p['sg_b2']),
        p['q_w1'], r2(p['q_b1']), r2(p['q_g1']), r2(p['q_be1']),
        p['q_w2'], r2(p['q_b2']), r2(p['q_g2']), r2(p['q_be2']),
        p['k_w1'], r2(p['k_b1']), r2(p['k_g1']), r2(p['k_be1']),
        p['k_w2'], r2(p['k_b2']), r2(p['k_g2']), r2(p['k_be2']),
        p['bt_w1'], r2(p['bt_b1']), r2(p['bt_g1']), r2(p['bt_be1']),
        p['bt_w2'], r2(p['bt_b2']), r2(p['bt_g2']), r2(p['bt_be2']),
        p['bt_w3'], r2(p['bt_b3']),
    ]
    seg, q, k, bt, lg = pl.pallas_call(
        _heads_body,
        grid=(N // _BNU,),
        in_specs=[pl.BlockSpec((_BNU, 128), lambda i: (i, 0)),
                  pl.BlockSpec((_BNU, 1), lambda i: (i, 0)),
                  pl.BlockSpec((16, 128), lambda i: (0, 0)),
                  pl.BlockSpec((1, 16), lambda i: (0, 0))]
                 + [_bspec(a) for a in consts_h],
        out_specs=[pl.BlockSpec((_BNU, 25), lambda i: (i, 0)),
                   pl.BlockSpec((_BNU, 256), lambda i: (i, 0)),
                   pl.BlockSpec((_BNU, 256), lambda i: (i, 0)),
                   pl.BlockSpec((_BNU, 1), lambda i: (i, 0)),
                   pl.BlockSpec((_BNU, 256), lambda i: (i, 0))],
        out_shape=[jax.ShapeDtypeStruct((N, 25), jnp.float32),
                   jax.ShapeDtypeStruct((N, 256), jnp.float32),
                   jax.ShapeDtypeStruct((N, 256), jnp.float32),
                   jax.ShapeDtypeStruct((N, 1), jnp.float32),
                   jax.ShapeDtypeStruct((N, 256), jnp.float32)],
    )(hc2, ids2, sums, cnts, *consts_h)

    inst = _inst_head(q, k, src, dst)
    return (seg, inst, bt, lg)
